# Initial kernel scaffold; baseline (speedup 1.0000x reference)
import jax
import jax.numpy as jnp
from jax.experimental import pallas as pl


def kernel(u, features, bias, W, in_w, in_b, s1_w, s1_b, s2_w, s2_b, x0, a_bar0):
    raise NotImplementedError("write your pallas kernel here")


# speedup vs baseline: 1.8428x; 1.8428x over previous
import jax
import jax.numpy as jnp
from jax.experimental import pallas as pl

N = 256
D = 32
K_NEIGHBORS = 50
NSTEPS = 20
GAMMA = 0.1
DT = 0.05
BATCH = 4
H1 = 16

_PREC = jax.lax.Precision.HIGHEST
_F32 = jnp.float32


def _body(u_ref, feat_ref, bias_ref, in_wT_ref, in_b_ref, s1_row_ref,
          s1_b_ref, s2_wT_ref, s2_b_ref, x0_ref, out_ref):
    feat = feat_ref[...]
    nrm = jnp.sqrt(jnp.sum(feat * feat, axis=1, keepdims=True))
    fn = feat / jnp.maximum(nrm, 1e-12)
    sim = jax.lax.dot_general(fn, fn, (((1,), (1,)), ((), ())),
                              precision=_PREC, preferred_element_type=_F32)

    col_ids = jax.lax.broadcasted_iota(jnp.int32, (N, N), 1)
    neg_inf = _F32(-jnp.inf)

    def extract(_, carry):
        rem, acc = carry
        m = jnp.max(rem, axis=1, keepdims=True)
        is_max = rem == m
        first = jnp.min(jnp.where(is_max, col_ids, N), axis=1, keepdims=True)
        sel = col_ids == first
        acc = jnp.where(sel, rem, acc)
        rem = jnp.where(sel, neg_inf, rem)
        return rem, acc

    _, nmat = jax.lax.fori_loop(
        0, K_NEIGHBORS, extract, (sim, jnp.zeros((N, N), _F32)))

    in_b = in_b_ref[...]
    s1_row = s1_row_ref[...]
    s1_b = s1_b_ref[...]
    s2_wT = s2_wT_ref[...]
    s2_b = s2_b_ref[...]
    bias = bias_ref[...]

    const = []
    xs = []
    for b in range(BATCH):
        up = jax.lax.dot_general(u_ref[b], in_wT_ref[...],
                                 (((1,), (0,)), ((), ())),
                                 precision=_PREC, preferred_element_type=_F32)
        const.append(up + in_b + bias)
        xs.append(x0_ref[b])

    inv_sqrt2 = _F32(0.7071067811865476)

    def step(_, xs):
        a_cols = [jnp.tanh(jnp.sqrt(jnp.sum(x * x, axis=1, keepdims=True)
                                    + 1e-12)) for x in xs]
        a_mat = jnp.concatenate(a_cols, axis=1)
        syn = jax.lax.dot_general(nmat, a_mat, (((1,), (0,)), ((), ())),
                                  precision=_PREC, preferred_element_type=_F32)
        new_xs = []
        for b in range(BATCH):
            h = syn[:, b:b + 1]
            pre = h * s1_row + s1_b
            h1 = 0.5 * pre * (1.0 + jax.lax.erf(pre * inv_sqrt2))
            sig = jax.lax.dot_general(h1, s2_wT, (((1,), (0,)), ((), ())),
                                      precision=_PREC,
                                      preferred_element_type=_F32) + s2_b
            dx = sig + const[b] - GAMMA * xs[b]
            new_xs.append(xs[b] + dx * DT)
        return tuple(new_xs)

    xs = jax.lax.fori_loop(0, NSTEPS, step, tuple(xs))
    for b in range(BATCH):
        out_ref[b] = xs[b]


def kernel(u, features, bias, W, in_w, in_b, s1_w, s1_b, s2_w, s2_b, x0,
           a_bar0):
    del W, a_bar0
    in_wT = in_w.T
    s1_row = s1_w.reshape(1, H1)
    s2_wT = s2_w.T
    return pl.pallas_call(
        _body,
        out_shape=jax.ShapeDtypeStruct((BATCH, N, D), jnp.float32),
    )(u, features, bias, in_wT, in_b.reshape(1, D), s1_row,
      s1_b.reshape(1, H1), s2_wT, s2_b.reshape(1, D), x0)


# speedup vs baseline: 2.1360x; 1.1591x over previous
import jax
import jax.numpy as jnp
from jax.experimental import pallas as pl

N = 256
D = 32
K_NEIGHBORS = 50
NSTEPS = 20
GAMMA = 0.1
DT = 0.05
BATCH = 4
H1 = 16
BD = BATCH * D
BH = BATCH * H1

_PREC = jax.lax.Precision.HIGHEST
_F32 = jnp.float32
_I32 = jnp.int32


def _body(u_ref, feat_ref, bias_ref, in_wT_ref, in_b_ref, m1_ref, s1b_ref,
          m2_ref, s2b_ref, mred_ref, x0_ref, out_ref):
    feat = feat_ref[...]
    nrm = jnp.sqrt(jnp.sum(feat * feat, axis=1, keepdims=True))
    fn = feat / jnp.maximum(nrm, 1e-12)
    sim = jax.lax.dot_general(fn, fn, (((1,), (1,)), ((), ())),
                              precision=_PREC, preferred_element_type=_F32)

    ib = jax.lax.bitcast_convert_type(sim, _I32)
    skey = jnp.where(ib < 0, jnp.bitwise_xor(~ib, _I32(-2147483648)), ib)

    lo0 = jnp.full((N, 1), -(2 ** 30), _I32)
    hi0 = jnp.full((N, 1), 2 ** 30 - 1, _I32)

    def bisect_val(_, carry):
        lo, hi = carry
        mid = lo + jax.lax.shift_right_logical(hi - lo, 1)
        cnt = jnp.sum(jnp.where(skey >= mid, _I32(1), _I32(0)),
                      axis=1, keepdims=True)
        ge = cnt >= K_NEIGHBORS
        return jnp.where(ge, mid, lo), jnp.where(ge, hi, mid)

    v50, _ = jax.lax.fori_loop(0, 31, bisect_val, (lo0, hi0))

    gt = skey > v50
    tie = skey == v50
    need = K_NEIGHBORS - jnp.sum(jnp.where(gt, _I32(1), _I32(0)),
                                 axis=1, keepdims=True)
    col_ids = jax.lax.broadcasted_iota(_I32, (N, N), 1)

    def bisect_col(_, carry):
        lo, hi = carry
        mid = lo + jax.lax.shift_right_logical(hi - lo, 1)
        cnt = jnp.sum(jnp.where(tie & (col_ids < mid), _I32(1), _I32(0)),
                      axis=1, keepdims=True)
        ge = cnt >= need
        return jnp.where(ge, lo, mid), jnp.where(ge, mid, hi)

    _, cstar = jax.lax.fori_loop(
        0, 8, bisect_col,
        (jnp.zeros((N, 1), _I32), jnp.full((N, 1), N, _I32)))

    nmat = jnp.where(gt | (tie & (col_ids < cstar)), sim, _F32(0.0))

    in_wT = in_wT_ref[...]
    ups = [jax.lax.dot_general(u_ref[b], in_wT, (((1,), (0,)), ((), ())),
                               precision=_PREC, preferred_element_type=_F32)
           for b in range(BATCH)]
    bias = bias_ref[...]
    const = jnp.concatenate([up + bias for up in ups], axis=1) + in_b_ref[...]

    m1 = m1_ref[...]
    s1b = s1b_ref[...]
    m2 = m2_ref[...]
    s2b = s2b_ref[...]
    mred = mred_ref[...]

    x_init = jnp.concatenate([x0_ref[b] for b in range(BATCH)], axis=1)

    inv_sqrt2 = _F32(0.7071067811865476)

    def step(_, x):
        sq = jax.lax.dot_general(x * x, mred, (((1,), (0,)), ((), ())),
                                 precision=_PREC,
                                 preferred_element_type=_F32)
        amat = jnp.tanh(jnp.sqrt(sq + 1e-12))
        syn = jax.lax.dot_general(nmat, amat, (((1,), (0,)), ((), ())),
                                  precision=_PREC,
                                  preferred_element_type=_F32)
        pre = jax.lax.dot_general(syn, m1, (((1,), (0,)), ((), ())),
                                  precision=_PREC,
                                  preferred_element_type=_F32) + s1b
        h1 = 0.5 * pre * (1.0 + jax.lax.erf(pre * inv_sqrt2))
        sig = jax.lax.dot_general(h1, m2, (((1,), (0,)), ((), ())),
                                  precision=_PREC,
                                  preferred_element_type=_F32) + s2b
        return x + (sig + const - GAMMA * x) * DT

    x = jax.lax.fori_loop(0, NSTEPS, step, x_init)
    for b in range(BATCH):
        out_ref[b] = x[:, b * D:(b + 1) * D]


def kernel(u, features, bias, W, in_w, in_b, s1_w, s1_b, s2_w, s2_b, x0,
           a_bar0):
    del W, a_bar0
    eye_b = jnp.eye(BATCH, dtype=jnp.float32)
    m1 = jnp.kron(eye_b, s1_w.reshape(1, H1))
    m2 = jnp.kron(eye_b, s2_w.T)
    mred = jnp.kron(eye_b, jnp.ones((D, 1), jnp.float32))
    return pl.pallas_call(
        _body,
        out_shape=jax.ShapeDtypeStruct((BATCH, N, D), jnp.float32),
    )(u, features, bias, in_w.T, jnp.tile(in_b, BATCH).reshape(1, BD),
      m1, jnp.tile(s1_b, BATCH).reshape(1, BH), m2,
      jnp.tile(s2_b, BATCH).reshape(1, BD), mred, x0)
